# baseline (speedup 1.0000x reference)
import functools

import jax
import jax.numpy as jnp
from jax import lax
from jax.experimental import pallas as pl
from jax.experimental.pallas import tpu as pltpu
from jax.experimental.pallas import tpu_sc as plsc

B = 16384
D = 64
CH = 128


def _sc_gather(P_table, Q_table, uidx, pidx):
    info = plsc.get_sparse_core_info()
    NC, NS = info.num_cores, info.num_subcores
    NW = NC * NS
    bpw = B // NW
    nch = bpw // CH
    mesh = plsc.VectorSubcoreMesh(core_axis_name="c", subcore_axis_name="s")

    u3 = uidx.reshape(NW, nch, CH)
    p3 = pidx.reshape(NW, nch, CH)

    @functools.partial(
        pl.kernel,
        mesh=mesh,
        compiler_params=pltpu.CompilerParams(use_tc_tiling_on_sc=False),
        out_type=[
            jax.ShapeDtypeStruct((B, D), jnp.float32),
            jax.ShapeDtypeStruct((B, D), jnp.float32),
        ],
        scratch_types=[
            pltpu.VMEM((nch, CH), jnp.int32),
            pltpu.VMEM((nch, CH), jnp.int32),
            pltpu.VMEM((bpw, D), jnp.float32),
            pltpu.VMEM((bpw, D), jnp.float32),
            pltpu.SemaphoreType.DMA,
        ],
    )
    def k(P_hbm, Q_hbm, u_hbm, pr_hbm, Pout, Qout, uv, pv, Pr, Qr, sem):
        wid = lax.axis_index("s") * NC + lax.axis_index("c")
        base = wid * bpw
        pltpu.sync_copy(u_hbm.at[wid], uv)
        pltpu.sync_copy(pr_hbm.at[wid], pv)
        copies = []
        for j in range(nch):
            copies.append(
                pltpu.async_copy(P_hbm.at[uv.at[j]], Pr.at[pl.ds(j * CH, CH)], sem)
            )
            copies.append(
                pltpu.async_copy(Q_hbm.at[pv.at[j]], Qr.at[pl.ds(j * CH, CH)], sem)
            )
        for c in copies:
            c.wait()
        pltpu.sync_copy(Pr, Pout.at[pl.ds(base, bpw)])
        pltpu.sync_copy(Qr, Qout.at[pl.ds(base, bpw)])

    return k(P_table, Q_table, u3, p3)


def _mlp_body(p, q, w1a, w1b, b1, w2, b2, o):
    h = jnp.dot(p[...], w1a[...], preferred_element_type=jnp.float32)
    h = h + jnp.dot(q[...], w1b[...], preferred_element_type=jnp.float32)
    h = jnp.maximum(h + b1[...], 0.0)
    o[...] = jnp.sum(h * w2[...], axis=1, keepdims=True) + b2[...]


def _tc_mlp(P, Q, W1a, W1b, b1r, w2r, b2r):
    TB = 2048
    return pl.pallas_call(
        _mlp_body,
        grid=(B // TB,),
        in_specs=[
            pl.BlockSpec((TB, D), lambda i: (i, 0)),
            pl.BlockSpec((TB, D), lambda i: (i, 0)),
            pl.BlockSpec((D, D), lambda i: (0, 0)),
            pl.BlockSpec((D, D), lambda i: (0, 0)),
            pl.BlockSpec((1, D), lambda i: (0, 0)),
            pl.BlockSpec((1, D), lambda i: (0, 0)),
            pl.BlockSpec((1, 1), lambda i: (0, 0)),
        ],
        out_specs=pl.BlockSpec((TB, 1), lambda i: (i, 0)),
        out_shape=jax.ShapeDtypeStruct((B, 1), jnp.float32),
    )(P, Q, W1a, W1b, b1r, w2r, b2r)


def kernel(user, product, P_table, Q_table, W1, b1, W2, b2):
    user = user.astype(jnp.int32)
    product = product.astype(jnp.int32)
    P, Q = _sc_gather(P_table, Q_table, user, product)
    W1a = W1[:D]
    W1b = W1[D:]
    return _tc_mlp(
        P,
        Q,
        W1a,
        W1b,
        b1.reshape(1, D),
        W2.reshape(1, D),
        b2.reshape(1, 1),
    )
